# Initial kernel scaffold; baseline (speedup 1.0000x reference)
import jax
import jax.numpy as jnp
from jax.experimental import pallas as pl


def kernel(x, edge_index, W1, b1, g1, be1, W2, b2, W3, b3, W4, b4):
    raise NotImplementedError("write your pallas kernel here")


# speedup vs baseline: 11.8507x; 11.8507x over previous
import functools

import jax
import jax.numpy as jnp
from jax import lax
from jax.experimental import pallas as pl
from jax.experimental.pallas import tpu as pltpu
from jax.experimental.pallas import tpu_sc as plsc

N = 10000
H = 128
E = 320000

NC = 2
NS = 16
NW = NC * NS

NP = 10240
CH = 128
NCHUNK = 79
EPT = NCHUNK * CH
EP = NW * EPT
ROWS_PER_TILE = NP // NS

BLK = 128
GRID = NP // BLK

_MESH = plsc.VectorSubcoreMesh(core_axis_name="c", subcore_axis_name="s")


def _make_prop_wide():
    @functools.partial(
        pl.kernel,
        out_type=jax.ShapeDtypeStruct((NC, NP, H), jnp.float32),
        mesh=_MESH,
        scratch_types=[
            pltpu.VMEM((NCHUNK, CH), jnp.int32),
            pltpu.VMEM((NCHUNK, CH), jnp.int32),
            pltpu.VMEM((CH, H), jnp.float32),
            pltpu.VMEM((64, H), jnp.float32),
            pltpu.VMEM_SHARED((NP, H), jnp.float32),
            pltpu.SemaphoreType.DMA,
        ],
    )
    def prop(y_hbm, row_hbm, col_hbm, out_hbm, ridx, cidx, rows, zbuf, acc, sem):
        cid = lax.axis_index("c")
        sid = lax.axis_index("s")
        wid = sid * NC + cid

        def zrow(r, _):
            for j in range(H // 16):
                zbuf[r, pl.ds(j * 16, 16)] = jnp.zeros((16,), jnp.float32)
            return 0
        lax.fori_loop(0, 64, zrow, 0)
        base_r = sid * ROWS_PER_TILE
        for k in range(ROWS_PER_TILE // 64):
            pltpu.sync_copy(zbuf, acc.at[pl.ds(base_r + k * 64, 64)])
        plsc.subcore_barrier()

        pltpu.sync_copy(row_hbm.at[wid], ridx)
        pltpu.sync_copy(col_hbm.at[wid], cidx)

        def chunk(c, _):
            pltpu.async_copy(y_hbm.at[ridx.at[c]], rows, sem).wait()
            pltpu.sync_copy(rows, acc.at[cidx.at[c]], add=True)
            return 0
        lax.fori_loop(0, NCHUNK, chunk, 0)

        plsc.subcore_barrier()
        pltpu.sync_copy(acc.at[pl.ds(base_r, ROWS_PER_TILE)],
                        out_hbm.at[cid, pl.ds(base_r, ROWS_PER_TILE)])

    return prop


def _make_prop_scalar(gather: bool):
    scratch = [
        pltpu.VMEM((NCHUNK, CH), jnp.int32),
        pltpu.VMEM((NP,), jnp.float32),
        pltpu.SemaphoreType.DMA,
    ]
    if gather:
        scratch = [pltpu.VMEM((NCHUNK, CH), jnp.int32),
                   pltpu.VMEM((NP,), jnp.float32),
                   ] + scratch

    @functools.partial(
        pl.kernel,
        out_type=jax.ShapeDtypeStruct((NW, NP), jnp.float32),
        mesh=_MESH,
        scratch_types=scratch,
        compiler_params=pltpu.CompilerParams(needs_layout_passes=False),
    )
    def prop(*args):
        if gather:
            y_hbm, row_hbm, col_hbm, out_hbm, ridx, ytab, cidx, accl, sem = args
        else:
            col_hbm, out_hbm, cidx, accl, sem = args
        cid = lax.axis_index("c")
        sid = lax.axis_index("s")
        wid = sid * NC + cid

        def zvec(r, _):
            accl[pl.ds(r * 16, 16)] = jnp.zeros((16,), jnp.float32)
            return 0
        lax.fori_loop(0, NP // 16, zvec, 0)

        if gather:
            pltpu.sync_copy(y_hbm, ytab)
            pltpu.sync_copy(row_hbm.at[wid], ridx)
        pltpu.sync_copy(col_hbm.at[wid], cidx)

        ones = jnp.ones((16,), jnp.float32)

        def chunk(c, _):
            for j in range(CH // 16):
                cv = cidx[c, pl.ds(j * 16, 16)]
                if gather:
                    rv = ridx[c, pl.ds(j * 16, 16)]
                    vals = plsc.load_gather(ytab, [rv])
                else:
                    vals = ones
                plsc.addupdate_scatter(accl, [cv], vals)
            return 0
        lax.fori_loop(0, NCHUNK, chunk, 0)

        pltpu.sync_copy(accl, out_hbm.at[wid])

    return prop


_prop_wide = _make_prop_wide()
_prop_deg = _make_prop_scalar(gather=False)
_prop_sc1 = _make_prop_scalar(gather=True)


def _spec(r, c):
    return pl.BlockSpec((r, c), lambda i: (i, 0))


def _wspec(r, c):
    return pl.BlockSpec((r, c), lambda i: (0, 0))


def _kdeg_body(degp_ref, dinv_ref):
    deg = jnp.sum(degp_ref[...], axis=0) + 1.0
    dinv_ref[...] = lax.rsqrt(deg)


def _dense_deg(degp):
    return pl.pallas_call(
        _kdeg_body,
        out_shape=jax.ShapeDtypeStruct((NP,), jnp.float32),
    )(degp)


def _k1_body(x_ref, w_ref, dinv_ref, y_ref):
    xw = jnp.dot(x_ref[...], w_ref[...], preferred_element_type=jnp.float32)
    y_ref[...] = dinv_ref[...] * xw


def _dense_first(xp, W1, dinv_col):
    return pl.pallas_call(
        _k1_body,
        grid=(GRID,),
        in_specs=[_spec(BLK, H), _wspec(H, H), _spec(BLK, 1)],
        out_specs=_spec(BLK, H),
        out_shape=jax.ShapeDtypeStruct((NP, H), jnp.float32),
    )(xp, W1, dinv_col)


def _make_mid_body(use_ln):
    def body(a0_ref, a1_ref, y_ref, dinv_ref, b_ref, g_ref, be_ref, w_ref, out_ref):
        dinv = dinv_ref[...]
        t = dinv * (a0_ref[...] + a1_ref[...] + y_ref[...]) + b_ref[:1, :]
        if use_ln:
            mu = jnp.mean(t, axis=-1, keepdims=True)
            var = jnp.mean((t - mu) ** 2, axis=-1, keepdims=True)
            t = (t - mu) * lax.rsqrt(var + 1e-5) * g_ref[:1, :] + be_ref[:1, :]
        h = jnp.maximum(t, 0.0)
        hw = jnp.dot(h, w_ref[...], preferred_element_type=jnp.float32)
        out_ref[...] = dinv * hw
    return body


def _dense_mid(a, y, dinv_col, b2d, g2d, be2d, Wn, use_ln, wcols):
    return pl.pallas_call(
        _make_mid_body(use_ln),
        grid=(GRID,),
        in_specs=[_spec(BLK, H), _spec(BLK, H), _spec(BLK, H), _spec(BLK, 1),
                  _wspec(8, H), _wspec(8, H), _wspec(8, H), _wspec(H, wcols)],
        out_specs=_spec(BLK, wcols),
        out_shape=jax.ShapeDtypeStruct((NP, wcols), jnp.float32),
    )(a[0], a[1], y, dinv_col, b2d, g2d, be2d, Wn)


def _k5_body(parts_ref, y_ref, dinv_ref, b_ref, out_ref):
    acc = jnp.sum(parts_ref[...], axis=0)
    out_ref[...] = dinv_ref[...] * (acc + y_ref[...]) + b_ref[...]


def _dense_last(parts, y4f, dinvf, b4v):
    return pl.pallas_call(
        _k5_body,
        out_shape=jax.ShapeDtypeStruct((NP,), jnp.float32),
    )(parts, y4f, dinvf, b4v)


def kernel(x, edge_index, W1, b1, g1, be1, W2, b2, W3, b3, W4, b4):
    f32 = jnp.float32
    row = edge_index[0].astype(jnp.int32)
    col = edge_index[1].astype(jnp.int32)
    pad = EP - E
    dummy = jnp.full((pad,), N, jnp.int32)
    row3d = jnp.concatenate([row, dummy]).reshape(NW, NCHUNK, CH)
    col3d = jnp.concatenate([col, dummy]).reshape(NW, NCHUNK, CH)

    xp = jnp.zeros((NP, H), f32).at[:N].set(x.astype(f32))
    b1_2d = jnp.broadcast_to(b1.reshape(1, H), (8, H))
    b2_2d = jnp.broadcast_to(b2.reshape(1, H), (8, H))
    b3_2d = jnp.broadcast_to(b3.reshape(1, H), (8, H))
    g1_2d = jnp.broadcast_to(g1.reshape(1, H), (8, H))
    be1_2d = jnp.broadcast_to(be1.reshape(1, H), (8, H))
    zeros_2d = jnp.zeros((8, H), f32)
    W4rep = jnp.broadcast_to(W4.reshape(H, 1), (H, 16)).astype(f32)
    b4v = jnp.broadcast_to(b4.reshape(1), (NP,)).astype(f32)

    degp = _prop_deg(col3d)
    dinvf = _dense_deg(degp)
    dinv_col = dinvf.reshape(NP, 1)

    y1 = _dense_first(xp, W1.astype(f32), dinv_col)

    a1 = _prop_wide(y1, row3d, col3d)
    y2 = _dense_mid(a1, y1, dinv_col, b1_2d, g1_2d, be1_2d,
                    W2.astype(f32), True, H)

    a2 = _prop_wide(y2, row3d, col3d)
    y3 = _dense_mid(a2, y2, dinv_col, b2_2d, zeros_2d, zeros_2d,
                    W3.astype(f32), False, H)

    a3 = _prop_wide(y3, row3d, col3d)
    y4_16 = _dense_mid(a3, y3, dinv_col, b3_2d, zeros_2d, zeros_2d,
                       W4rep, False, 16)
    y4f = y4_16[:, 0]

    a4 = _prop_sc1(y4f, row3d, col3d)
    res = _dense_last(a4, y4f, dinvf, b4v)

    return res[:N]
